# baseline (speedup 1.0000x reference)
import functools

import jax
import jax.numpy as jnp
from jax import lax
from jax.experimental import pallas as pl
from jax.experimental.pallas import tpu as pltpu
from jax.experimental.pallas import tpu_sc as plsc

N = 320000
D = 128
H = 128
OUT = 64
C = 16000

NC = 2
NS = 16
NW = NC * NS
CPW = C // NW

RB = 320
TA = 512
TB = 256

_NEG = -3.0e38


def _mlp_ln_body(x_ref, cl_ref, w1_ref, b1_ref, w2_ref, b2_ref, g_ref,
                 be_ref, o_ref, bnd_ref, cnt_ref):
    i = pl.program_id(0)

    @pl.when(i == 0)
    def _():
        cnt_ref[...] = jnp.zeros_like(cnt_ref)

    h = jnp.dot(x_ref[...], w1_ref[...], preferred_element_type=jnp.float32)
    h = h + b1_ref[...]
    h = h * jax.nn.sigmoid(h)
    o = jnp.dot(h, w2_ref[...], preferred_element_type=jnp.float32)
    o = o + b2_ref[...]
    ones_w = jnp.full((OUT, 128), 1.0 / OUT, jnp.float32)
    mu = jnp.dot(o, ones_w, preferred_element_type=jnp.float32)[:, :OUT]
    e2 = jnp.dot(o * o, ones_w, preferred_element_type=jnp.float32)[:, :OUT]
    var = e2 - mu * mu
    o = (o - mu) * jax.lax.rsqrt(var + 1e-5) * g_ref[...] + be_ref[...]
    o_ref[...] = jnp.concatenate([o, jnp.zeros_like(o)], axis=1)

    cl = cl_ref[0, 0, :]
    thr = lax.broadcasted_iota(jnp.int32, (1, 128), 1) * CPW
    cmp = (cl[:, None] < thr).astype(jnp.int32)
    cnt_ref[...] += jnp.sum(cmp, axis=0, keepdims=True)

    @pl.when(i == pl.num_programs(0) - 1)
    def _():
        bnd_ref[...] = cnt_ref[...]


def _tc_mlp_ln(x, cl3d, W1, b1, W2, b2, gamma, beta):
    grid = N // RB
    return pl.pallas_call(
        _mlp_ln_body,
        grid=(grid,),
        in_specs=[
            pl.BlockSpec((RB, D), lambda i: (i, 0)),
            pl.BlockSpec((1, 1, RB), lambda i: (i, 0, 0)),
            pl.BlockSpec((D, H), lambda i: (0, 0)),
            pl.BlockSpec((1, H), lambda i: (0, 0)),
            pl.BlockSpec((H, OUT), lambda i: (0, 0)),
            pl.BlockSpec((1, OUT), lambda i: (0, 0)),
            pl.BlockSpec((1, OUT), lambda i: (0, 0)),
            pl.BlockSpec((1, OUT), lambda i: (0, 0)),
        ],
        out_specs=[
            pl.BlockSpec((RB, 2 * OUT), lambda i: (i, 0)),
            pl.BlockSpec((1, 128), lambda i: (0, 0)),
        ],
        out_shape=[
            jax.ShapeDtypeStruct((N, 2 * OUT), jnp.float32),
            jax.ShapeDtypeStruct((1, 128), jnp.int32),
        ],
        scratch_shapes=[pltpu.VMEM((1, 128), jnp.int32)],
    )(x, cl3d, W1, b1, W2, b2, gamma, beta)


def _segmax_body(o_hbm, cl_hbm, bnd_hbm, aggr_hbm, bnd_v, cl_v, o_v, acc,
                 sem_c, sem_o):
    cid = lax.axis_index("c")
    sid = lax.axis_index("s")
    w = sid * NC + cid
    base = w * CPW

    pltpu.sync_copy(bnd_hbm, bnd_v)
    bv = bnd_v[pl.ds(w, 16)]
    start = bv[0]
    end = bv[1]

    neg = jnp.full((16,), _NEG, jnp.float32)

    def init_body(i, carry):
        acc[pl.ds(i * 16, 16)] = neg
        return carry

    lax.fori_loop(0, (CPW + 2) * OUT // 16, init_body, 0)

    astart = (start // 8) * 8
    ntiles = (end - astart + TA - 1) // TA

    def row0_of(t):
        return jnp.minimum(astart + t * TA, N - TA)

    def stage(t):
        b = lax.rem(t, 2)
        row0 = row0_of(t)
        pltpu.async_copy(cl_hbm.at[pl.ds(row0, TA)], cl_v.at[b], sem_c.at[b])
        pltpu.async_copy(o_hbm.at[pl.ds(row0, TA), pl.ds(0, OUT)],
                         o_v.at[b], sem_o.at[b])

    def wait_stage(t):
        b = lax.rem(t, 2)
        row0 = row0_of(t)
        pltpu.make_async_copy(cl_hbm.at[pl.ds(row0, TA)], cl_v.at[b],
                              sem_c.at[b]).wait()
        pltpu.make_async_copy(o_hbm.at[pl.ds(row0, TA), pl.ds(0, OUT)],
                              o_v.at[b], sem_o.at[b]).wait()

    def flush(cur_c, ms):
        off = cur_c * OUT
        for j in range(OUT // 16):
            a = acc[pl.ds(off + 16 * j, 16)]
            acc[pl.ds(off + 16 * j, 16)] = jnp.maximum(a, ms[j])

    @pl.when(ntiles > 0)
    def _():
        stage(0)

        def tile_body(t, carry):
            @pl.when(t + 1 < ntiles)
            def _():
                stage(t + 1)

            wait_stage(t)
            b = lax.rem(t, 2)

            def grp_body(g, gc):
                cur_c, ms = gc
                r0 = g * 16
                cv = cl_v[b, pl.ds(r0, 16)]
                clocv = jnp.clip(cv - base, -1, CPW) + 1
                for k in range(16):
                    cloc = clocv[k]
                    vs = [o_v[b, r0 + k, pl.ds(16 * j, 16)]
                          for j in range(OUT // 16)]
                    fl = cloc != cur_c

                    @pl.when(fl)
                    def _(cur_c=cur_c, ms=ms):
                        flush(cur_c, ms)

                    ms = [jnp.where(fl, vs[j], jnp.maximum(ms[j], vs[j]))
                          for j in range(OUT // 16)]
                    cur_c = jnp.where(fl, cloc, cur_c)
                return (cur_c, ms)

            return lax.fori_loop(0, TA // 16, grp_body, carry)

        carry0 = (jnp.int32(0), [neg for _ in range(OUT // 16)])
        cur_c, ms = lax.fori_loop(0, ntiles, tile_body, carry0)
        flush(cur_c, ms)

    pltpu.sync_copy(acc.at[pl.ds(OUT, CPW * OUT)],
                    aggr_hbm.at[pl.ds(base * OUT, CPW * OUT)])


def _sc_segmax(o_flat, cl, bnd):
    mesh = plsc.VectorSubcoreMesh(core_axis_name="c", subcore_axis_name="s",
                                  num_cores=NC, num_subcores=NS)
    f = functools.partial(
        pl.kernel,
        out_type=jax.ShapeDtypeStruct((C * OUT,), jnp.float32),
        mesh=mesh,
        scratch_types=[
            pltpu.VMEM((128,), jnp.int32),
            pltpu.VMEM((2, TA), jnp.int32),
            pltpu.VMEM((2, TA, OUT), jnp.float32),
            pltpu.VMEM(((CPW + 2) * OUT,), jnp.float32),
            pltpu.SemaphoreType.DMA((2,)),
            pltpu.SemaphoreType.DMA((2,)),
        ],
        compiler_params=pltpu.CompilerParams(use_tc_tiling_on_sc=False),
    )(_segmax_body)
    return f(o_flat, cl, bnd)


def _rsqrt16(ss):
    i = lax.bitcast_convert_type(ss, jnp.int32)
    i = jnp.int32(0x5F3759DF) - lax.shift_right_logical(i, 1)
    y = lax.bitcast_convert_type(i, jnp.float32)
    for _ in range(2):
        y = y * (1.5 - 0.5 * ss * y * y)
    return y


def _gather_norm_body(aggr_hbm, o_hbm, cl_hbm, out_hbm, idx_v, g_v, o_v,
                      sem_c, sem_g, sem_o, sem_w):
    cid = lax.axis_index("c")
    sid = lax.axis_index("s")
    w = sid * NC + cid
    nt = N // TB
    nloop = (nt - w + NW - 1) // NW

    def row0_of(t):
        return (w + t * NW) * TB

    def fire_cl(t):
        b = lax.rem(t, 2)
        row0 = row0_of(t)
        for j in range(TB // 128):
            pltpu.async_copy(cl_hbm.at[pl.ds(row0 + 128 * j, 128)],
                             idx_v.at[b, j], sem_c.at[b])

    def wait_cl(t):
        b = lax.rem(t, 2)
        row0 = row0_of(t)
        for j in range(TB // 128):
            pltpu.make_async_copy(cl_hbm.at[pl.ds(row0 + 128 * j, 128)],
                                  idx_v.at[b, j], sem_c.at[b]).wait()

    def fire_go(t):
        b = lax.rem(t, 2)
        row0 = row0_of(t)

        @pl.when(t >= 2)
        def _():
            r2 = row0_of(t - 2)
            pltpu.make_async_copy(
                o_v.at[b], out_hbm.at[pl.ds(r2, TB), pl.ds(0, OUT)],
                sem_w.at[b]).wait()
            pltpu.make_async_copy(
                g_v.at[b], out_hbm.at[pl.ds(r2, TB), pl.ds(OUT, OUT)],
                sem_w.at[b]).wait()

        for j in range(TB // 128):
            pltpu.async_copy(aggr_hbm.at[idx_v.at[b, j]],
                             g_v.at[b, pl.ds(128 * j, 128)], sem_g.at[b])
        pltpu.async_copy(o_hbm.at[pl.ds(row0, TB), pl.ds(0, OUT)],
                         o_v.at[b], sem_o.at[b])

    def wait_go(t):
        b = lax.rem(t, 2)
        row0 = row0_of(t)
        for j in range(TB // 128):
            pltpu.make_async_copy(
                aggr_hbm.at[idx_v.at[b, j]],
                g_v.at[b, pl.ds(128 * j, 128)], sem_g.at[b]).wait()
        pltpu.make_async_copy(o_hbm.at[pl.ds(row0, TB), pl.ds(0, OUT)],
                              o_v.at[b], sem_o.at[b]).wait()

    def compute(t):
        b = lax.rem(t, 2)
        row0 = row0_of(t)

        def grp_body(g, rcarry):
            r0 = g * 4
            for k in range(4):
                r = r0 + k
                ov = [o_v[b, r, pl.ds(16 * j, 16)] for j in range(OUT // 16)]
                gv = [g_v[b, r, pl.ds(16 * j, 16)] for j in range(OUT // 16)]
                p = ov[0] * ov[0]
                for j in range(1, OUT // 16):
                    p = p + ov[j] * ov[j]
                for j in range(OUT // 16):
                    p = p + gv[j] * gv[j]
                lanes = lax.iota(jnp.int32, 16)
                for step in (1, 2, 4, 8):
                    p = p + p.at[lanes ^ step].get(mode="promise_in_bounds")
                ss = jnp.maximum(p, 1e-24)
                y = _rsqrt16(ss)
                for j in range(OUT // 16):
                    o_v[b, r, pl.ds(16 * j, 16)] = ov[j] * y
                    g_v[b, r, pl.ds(16 * j, 16)] = gv[j] * y
            return rcarry

        lax.fori_loop(0, TB // 4, grp_body, 0)
        pltpu.async_copy(o_v.at[b], out_hbm.at[pl.ds(row0, TB), pl.ds(0, OUT)],
                         sem_w.at[b])
        pltpu.async_copy(g_v.at[b],
                         out_hbm.at[pl.ds(row0, TB), pl.ds(OUT, OUT)],
                         sem_w.at[b])

    @pl.when(nloop > 0)
    def _():
        fire_cl(0)

        @pl.when(nloop > 1)
        def _():
            fire_cl(1)

        wait_cl(0)
        fire_go(0)

        def body(t, carry):
            wait_go(t)

            @pl.when(t + 2 < nloop)
            def _():
                fire_cl(t + 2)

            @pl.when(t + 1 < nloop)
            def _():
                wait_cl(t + 1)
                fire_go(t + 1)

            compute(t)
            return carry

        lax.fori_loop(0, nloop, body, 0)

        def drain(t):
            b = lax.rem(t, 2)
            pltpu.make_async_copy(
                o_v.at[b], out_hbm.at[pl.ds(row0_of(t), TB), pl.ds(0, OUT)],
                sem_w.at[b]).wait()
            pltpu.make_async_copy(
                g_v.at[b],
                out_hbm.at[pl.ds(row0_of(t), TB), pl.ds(OUT, OUT)],
                sem_w.at[b]).wait()

        @pl.when(nloop > 1)
        def _():
            drain(nloop - 2)

        drain(nloop - 1)


def _sc_gather_norm(aggr, o2, cl):
    mesh = plsc.VectorSubcoreMesh(core_axis_name="c", subcore_axis_name="s",
                                  num_cores=NC, num_subcores=NS)
    f = functools.partial(
        pl.kernel,
        out_type=jax.ShapeDtypeStruct((N, 2 * OUT), jnp.float32),
        mesh=mesh,
        scratch_types=[
            pltpu.VMEM((2, TB // 128, 128), jnp.int32),
            pltpu.VMEM((2, TB, OUT), jnp.float32),
            pltpu.VMEM((2, TB, OUT), jnp.float32),
            pltpu.SemaphoreType.DMA((2,)),
            pltpu.SemaphoreType.DMA((2,)),
            pltpu.SemaphoreType.DMA((2,)),
            pltpu.SemaphoreType.DMA((2,)),
        ],
        compiler_params=pltpu.CompilerParams(use_tc_tiling_on_sc=False),
    )(_gather_norm_body)
    return f(aggr, o2, cl)


def kernel(x, clusters, batch, W1, b1, W2, b2, gamma, beta):
    del batch
    x = x.astype(jnp.float32)
    cl = clusters.astype(jnp.int32)
    cl3d = cl.reshape(N // RB, 1, RB)
    o2, bnd = _tc_mlp_ln(x, cl3d, W1, b1.reshape(1, H), W2,
                         b2.reshape(1, OUT), gamma.reshape(1, OUT),
                         beta.reshape(1, OUT))
    aggr_flat = _sc_segmax(o2, cl, bnd.reshape(128))
    return _sc_gather_norm(aggr_flat.reshape(C, OUT), o2, cl)


# speedup vs baseline: 1.5696x; 1.5696x over previous
import functools

import jax
import jax.numpy as jnp
from jax import lax
from jax.experimental import pallas as pl
from jax.experimental.pallas import tpu as pltpu
from jax.experimental.pallas import tpu_sc as plsc

N = 320000
D = 128
H = 128
OUT = 64
C = 16000

NC = 2
NS = 16
NW = NC * NS
CPW = C // NW

RB = 1280
TA = 512
TB = 256

_NEG = -3.0e38


def _mlp_ln_body(x_ref, cl_ref, w1_ref, b1_ref, w2_ref, b2_ref, g_ref,
                 be_ref, o_ref, bnd_ref, cnt_ref):
    i = pl.program_id(0)

    @pl.when(i == 0)
    def _():
        cnt_ref[...] = jnp.zeros_like(cnt_ref)

    h = jnp.dot(x_ref[...], w1_ref[...], preferred_element_type=jnp.float32)
    h = h + b1_ref[...]
    h = h * jax.nn.sigmoid(h)
    o = jnp.dot(h, w2_ref[...], preferred_element_type=jnp.float32)
    o = o + b2_ref[...]
    ones_w = jnp.full((OUT, 128), 1.0 / OUT, jnp.float32)
    mu = jnp.dot(o, ones_w, preferred_element_type=jnp.float32)[:, :OUT]
    e2 = jnp.dot(o * o, ones_w, preferred_element_type=jnp.float32)[:, :OUT]
    var = e2 - mu * mu
    o = (o - mu) * jax.lax.rsqrt(var + 1e-5) * g_ref[...] + be_ref[...]
    o_ref[...] = jnp.concatenate([o, jnp.zeros_like(o)], axis=1)

    cl = cl_ref[0, 0, :]
    thr = lax.broadcasted_iota(jnp.int32, (1, 128), 1) * CPW
    cmp = (cl[:, None] < thr).astype(jnp.int32)
    cnt_ref[...] += jnp.sum(cmp, axis=0, keepdims=True)

    @pl.when(i == pl.num_programs(0) - 1)
    def _():
        bnd_ref[...] = cnt_ref[...]


def _tc_mlp_ln(x, cl3d, W1, b1, W2, b2, gamma, beta):
    grid = N // RB
    return pl.pallas_call(
        _mlp_ln_body,
        grid=(grid,),
        in_specs=[
            pl.BlockSpec((RB, D), lambda i: (i, 0)),
            pl.BlockSpec((1, 1, RB), lambda i: (i, 0, 0)),
            pl.BlockSpec((D, H), lambda i: (0, 0)),
            pl.BlockSpec((1, H), lambda i: (0, 0)),
            pl.BlockSpec((H, OUT), lambda i: (0, 0)),
            pl.BlockSpec((1, OUT), lambda i: (0, 0)),
            pl.BlockSpec((1, OUT), lambda i: (0, 0)),
            pl.BlockSpec((1, OUT), lambda i: (0, 0)),
        ],
        out_specs=[
            pl.BlockSpec((RB, 2 * OUT), lambda i: (i, 0)),
            pl.BlockSpec((1, 128), lambda i: (0, 0)),
        ],
        out_shape=[
            jax.ShapeDtypeStruct((N, 2 * OUT), jnp.float32),
            jax.ShapeDtypeStruct((1, 128), jnp.int32),
        ],
        scratch_shapes=[pltpu.VMEM((1, 128), jnp.int32)],
    )(x, cl3d, W1, b1, W2, b2, gamma, beta)


def _segmax_body(o_hbm, cl_hbm, bnd_hbm, aggr_hbm, bnd_v, cl_v, o_v, acc,
                 sem_c, sem_o):
    cid = lax.axis_index("c")
    sid = lax.axis_index("s")
    w = sid * NC + cid
    base = w * CPW

    pltpu.sync_copy(bnd_hbm, bnd_v)
    bv = bnd_v[pl.ds(w, 16)]
    start = bv[0]
    end = bv[1]

    neg = jnp.full((16,), _NEG, jnp.float32)

    def init_body(i, carry):
        acc[pl.ds(i * 16, 16)] = neg
        return carry

    lax.fori_loop(0, (CPW + 2) * OUT // 16, init_body, 0)

    astart = (start // 8) * 8
    ntiles = (end - astart + TA - 1) // TA

    def row0_of(t):
        return jnp.minimum(astart + t * TA, N - TA)

    def stage(t):
        b = lax.rem(t, 2)
        row0 = row0_of(t)
        pltpu.async_copy(cl_hbm.at[pl.ds(row0, TA)], cl_v.at[b], sem_c.at[b])
        pltpu.async_copy(o_hbm.at[pl.ds(row0, TA), pl.ds(0, OUT)],
                         o_v.at[b], sem_o.at[b])

    def wait_stage(t):
        b = lax.rem(t, 2)
        row0 = row0_of(t)
        pltpu.make_async_copy(cl_hbm.at[pl.ds(row0, TA)], cl_v.at[b],
                              sem_c.at[b]).wait()
        pltpu.make_async_copy(o_hbm.at[pl.ds(row0, TA), pl.ds(0, OUT)],
                              o_v.at[b], sem_o.at[b]).wait()

    def flush(cur_c, ms):
        off = cur_c * OUT
        for j in range(OUT // 16):
            a = acc[pl.ds(off + 16 * j, 16)]
            acc[pl.ds(off + 16 * j, 16)] = jnp.maximum(a, ms[j])

    @pl.when(ntiles > 0)
    def _():
        stage(0)

        def tile_body(t, carry):
            @pl.when(t + 1 < ntiles)
            def _():
                stage(t + 1)

            wait_stage(t)
            b = lax.rem(t, 2)

            def grp_body(g, gc):
                cur_c, ms = gc
                r0 = g * 16
                cv = cl_v[b, pl.ds(r0, 16)]
                clocv = jnp.clip(cv - base, -1, CPW) + 1
                for k in range(16):
                    cloc = clocv[k]
                    vs = [o_v[b, r0 + k, pl.ds(16 * j, 16)]
                          for j in range(OUT // 16)]
                    fl = cloc != cur_c

                    @pl.when(fl)
                    def _(cur_c=cur_c, ms=ms):
                        flush(cur_c, ms)

                    ms = [jnp.where(fl, vs[j], jnp.maximum(ms[j], vs[j]))
                          for j in range(OUT // 16)]
                    cur_c = jnp.where(fl, cloc, cur_c)
                return (cur_c, ms)

            return lax.fori_loop(0, TA // 16, grp_body, carry)

        carry0 = (jnp.int32(0), [neg for _ in range(OUT // 16)])
        cur_c, ms = lax.fori_loop(0, ntiles, tile_body, carry0)
        flush(cur_c, ms)

    pltpu.sync_copy(acc.at[pl.ds(OUT, CPW * OUT)],
                    aggr_hbm.at[pl.ds(base * OUT, CPW * OUT)])


def _sc_segmax(o_flat, cl, bnd):
    mesh = plsc.VectorSubcoreMesh(core_axis_name="c", subcore_axis_name="s",
                                  num_cores=NC, num_subcores=NS)
    f = functools.partial(
        pl.kernel,
        out_type=jax.ShapeDtypeStruct((C * OUT,), jnp.float32),
        mesh=mesh,
        scratch_types=[
            pltpu.VMEM((128,), jnp.int32),
            pltpu.VMEM((2, TA), jnp.int32),
            pltpu.VMEM((2, TA, OUT), jnp.float32),
            pltpu.VMEM(((CPW + 2) * OUT,), jnp.float32),
            pltpu.SemaphoreType.DMA((2,)),
            pltpu.SemaphoreType.DMA((2,)),
        ],
        compiler_params=pltpu.CompilerParams(use_tc_tiling_on_sc=False),
    )(_segmax_body)
    return f(o_flat, cl, bnd)


def _rsqrt16(ss):
    i = lax.bitcast_convert_type(ss, jnp.int32)
    i = jnp.int32(0x5F3759DF) - lax.shift_right_logical(i, 1)
    y = lax.bitcast_convert_type(i, jnp.float32)
    for _ in range(2):
        y = y * (1.5 - 0.5 * ss * y * y)
    return y


def _gather_norm_body(aggr_hbm, o_hbm, cl_hbm, out_hbm, idx_v, g_v, o_v,
                      sem_c, sem_g, sem_o, sem_w):
    cid = lax.axis_index("c")
    sid = lax.axis_index("s")
    w = sid * NC + cid
    nt = N // TB
    nloop = (nt - w + NW - 1) // NW

    def row0_of(t):
        return (w + t * NW) * TB

    def fire_cl(t):
        b = lax.rem(t, 2)
        row0 = row0_of(t)
        for j in range(TB // 128):
            pltpu.async_copy(cl_hbm.at[pl.ds(row0 + 128 * j, 128)],
                             idx_v.at[b, j], sem_c.at[b])

    def wait_cl(t):
        b = lax.rem(t, 2)
        row0 = row0_of(t)
        for j in range(TB // 128):
            pltpu.make_async_copy(cl_hbm.at[pl.ds(row0 + 128 * j, 128)],
                                  idx_v.at[b, j], sem_c.at[b]).wait()

    def fire_go(t):
        b = lax.rem(t, 2)
        row0 = row0_of(t)

        @pl.when(t >= 2)
        def _():
            r2 = row0_of(t - 2)
            pltpu.make_async_copy(
                o_v.at[b], out_hbm.at[pl.ds(r2, TB), pl.ds(0, OUT)],
                sem_w.at[b]).wait()
            pltpu.make_async_copy(
                g_v.at[b], out_hbm.at[pl.ds(r2, TB), pl.ds(OUT, OUT)],
                sem_w.at[b]).wait()

        for j in range(TB // 128):
            pltpu.async_copy(aggr_hbm.at[idx_v.at[b, j]],
                             g_v.at[b, pl.ds(128 * j, 128)], sem_g.at[b])
        pltpu.async_copy(o_hbm.at[pl.ds(row0, TB), pl.ds(0, OUT)],
                         o_v.at[b], sem_o.at[b])

    def wait_go(t):
        b = lax.rem(t, 2)
        row0 = row0_of(t)
        for j in range(TB // 128):
            pltpu.make_async_copy(
                aggr_hbm.at[idx_v.at[b, j]],
                g_v.at[b, pl.ds(128 * j, 128)], sem_g.at[b]).wait()
        pltpu.make_async_copy(o_hbm.at[pl.ds(row0, TB), pl.ds(0, OUT)],
                              o_v.at[b], sem_o.at[b]).wait()

    def compute(t):
        b = lax.rem(t, 2)
        row0 = row0_of(t)

        def grp_body(g, rcarry):
            r0 = g * 4
            for k in range(4):
                r = r0 + k
                ov = [o_v[b, r, pl.ds(16 * j, 16)] for j in range(OUT // 16)]
                gv = [g_v[b, r, pl.ds(16 * j, 16)] for j in range(OUT // 16)]
                p = ov[0] * ov[0]
                for j in range(1, OUT // 16):
                    p = p + ov[j] * ov[j]
                for j in range(OUT // 16):
                    p = p + gv[j] * gv[j]
                lanes = lax.iota(jnp.int32, 16)
                for step in (1, 2, 4, 8):
                    p = p + p.at[lanes ^ step].get(mode="promise_in_bounds")
                ss = jnp.maximum(p, 1e-24)
                y = _rsqrt16(ss)
                for j in range(OUT // 16):
                    o_v[b, r, pl.ds(16 * j, 16)] = ov[j] * y
                    g_v[b, r, pl.ds(16 * j, 16)] = gv[j] * y
            return rcarry

        lax.fori_loop(0, TB // 4, grp_body, 0)
        pltpu.async_copy(o_v.at[b], out_hbm.at[pl.ds(row0, TB), pl.ds(0, OUT)],
                         sem_w.at[b])
        pltpu.async_copy(g_v.at[b],
                         out_hbm.at[pl.ds(row0, TB), pl.ds(OUT, OUT)],
                         sem_w.at[b])

    @pl.when(nloop > 0)
    def _():
        fire_cl(0)

        @pl.when(nloop > 1)
        def _():
            fire_cl(1)

        wait_cl(0)
        fire_go(0)

        def body(t, carry):
            wait_go(t)

            @pl.when(t + 2 < nloop)
            def _():
                fire_cl(t + 2)

            @pl.when(t + 1 < nloop)
            def _():
                wait_cl(t + 1)
                fire_go(t + 1)

            compute(t)
            return carry

        lax.fori_loop(0, nloop, body, 0)

        def drain(t):
            b = lax.rem(t, 2)
            pltpu.make_async_copy(
                o_v.at[b], out_hbm.at[pl.ds(row0_of(t), TB), pl.ds(0, OUT)],
                sem_w.at[b]).wait()
            pltpu.make_async_copy(
                g_v.at[b],
                out_hbm.at[pl.ds(row0_of(t), TB), pl.ds(OUT, OUT)],
                sem_w.at[b]).wait()

        @pl.when(nloop > 1)
        def _():
            drain(nloop - 2)

        drain(nloop - 1)


def _sc_gather_norm(aggr, o2, cl):
    mesh = plsc.VectorSubcoreMesh(core_axis_name="c", subcore_axis_name="s",
                                  num_cores=NC, num_subcores=NS)
    f = functools.partial(
        pl.kernel,
        out_type=jax.ShapeDtypeStruct((N, 2 * OUT), jnp.float32),
        mesh=mesh,
        scratch_types=[
            pltpu.VMEM((2, TB // 128, 128), jnp.int32),
            pltpu.VMEM((2, TB, OUT), jnp.float32),
            pltpu.VMEM((2, TB, OUT), jnp.float32),
            pltpu.SemaphoreType.DMA((2,)),
            pltpu.SemaphoreType.DMA((2,)),
            pltpu.SemaphoreType.DMA((2,)),
            pltpu.SemaphoreType.DMA((2,)),
        ],
        compiler_params=pltpu.CompilerParams(use_tc_tiling_on_sc=False),
    )(_gather_norm_body)
    return f(aggr, o2, cl)


def kernel(x, clusters, batch, W1, b1, W2, b2, gamma, beta):
    del batch
    x = x.astype(jnp.float32)
    cl = clusters.astype(jnp.int32)
    cl3d = cl.reshape(N // RB, 1, RB)
    o2, bnd = _tc_mlp_ln(x, cl3d, W1, b1.reshape(1, H), W2,
                         b2.reshape(1, OUT), gamma.reshape(1, OUT),
                         beta.reshape(1, OUT))
    aggr_flat = _sc_segmax(o2, cl, bnd.reshape(128))
    return _sc_gather_norm(aggr_flat.reshape(C, OUT), o2, cl)


# speedup vs baseline: 1.7183x; 1.0947x over previous
import functools

import jax
import jax.numpy as jnp
from jax import lax
from jax.experimental import pallas as pl
from jax.experimental.pallas import tpu as pltpu
from jax.experimental.pallas import tpu_sc as plsc

N = 320000
D = 128
H = 128
OUT = 64
C = 16000

NC = 2
NS = 16
NW = NC * NS
CPW = C // NW

RB = 2560
TA = 512
TB = 256

_NEG = -3.0e38


def _mlp_ln_body(x_ref, cl_ref, w1_ref, b1_ref, w2_ref, b2_ref, g_ref,
                 be_ref, o_ref, bnd_ref, cnt_ref):
    i = pl.program_id(0)

    @pl.when(i == 0)
    def _():
        cnt_ref[...] = jnp.zeros_like(cnt_ref)

    h = jnp.dot(x_ref[...], w1_ref[...], preferred_element_type=jnp.float32)
    h = h + b1_ref[...]
    h = h * jax.nn.sigmoid(h)
    o = jnp.dot(h, w2_ref[...], preferred_element_type=jnp.float32)
    o = o + b2_ref[...]
    ones_w = jnp.full((OUT, 128), 1.0 / OUT, jnp.float32)
    mu = jnp.dot(o, ones_w, preferred_element_type=jnp.float32)[:, :OUT]
    e2 = jnp.dot(o * o, ones_w, preferred_element_type=jnp.float32)[:, :OUT]
    var = e2 - mu * mu
    o = (o - mu) * jax.lax.rsqrt(var + 1e-5) * g_ref[...] + be_ref[...]
    o_ref[...] = jnp.concatenate([o, jnp.zeros_like(o)], axis=1)

    cl = cl_ref[0, 0, :]
    thr = lax.broadcasted_iota(jnp.int32, (1, 128), 1) * CPW
    cmp = (cl[:, None] < thr).astype(jnp.int32)
    cnt_ref[...] += jnp.sum(cmp, axis=0, keepdims=True)

    @pl.when(i == pl.num_programs(0) - 1)
    def _():
        bnd_ref[...] = cnt_ref[...]


def _tc_mlp_ln(x, cl3d, W1, b1, W2, b2, gamma, beta):
    grid = N // RB
    return pl.pallas_call(
        _mlp_ln_body,
        grid=(grid,),
        in_specs=[
            pl.BlockSpec((RB, D), lambda i: (i, 0)),
            pl.BlockSpec((1, 1, RB), lambda i: (i, 0, 0)),
            pl.BlockSpec((D, H), lambda i: (0, 0)),
            pl.BlockSpec((1, H), lambda i: (0, 0)),
            pl.BlockSpec((H, OUT), lambda i: (0, 0)),
            pl.BlockSpec((1, OUT), lambda i: (0, 0)),
            pl.BlockSpec((1, OUT), lambda i: (0, 0)),
            pl.BlockSpec((1, OUT), lambda i: (0, 0)),
        ],
        out_specs=[
            pl.BlockSpec((RB, 2 * OUT), lambda i: (i, 0)),
            pl.BlockSpec((1, 128), lambda i: (0, 0)),
        ],
        out_shape=[
            jax.ShapeDtypeStruct((N, 2 * OUT), jnp.float32),
            jax.ShapeDtypeStruct((1, 128), jnp.int32),
        ],
        scratch_shapes=[pltpu.VMEM((1, 128), jnp.int32)],
    )(x, cl3d, W1, b1, W2, b2, gamma, beta)


def _segmax_body(o_hbm, cl_hbm, bnd_hbm, aggr_hbm, bnd_v, cl_v, o_v, acc,
                 sem_c, sem_o):
    cid = lax.axis_index("c")
    sid = lax.axis_index("s")
    w = sid * NC + cid
    base = w * CPW

    pltpu.sync_copy(bnd_hbm, bnd_v)
    bv = bnd_v[pl.ds(w, 16)]
    start = bv[0]
    end = bv[1]

    neg = jnp.full((16,), _NEG, jnp.float32)

    def init_body(i, carry):
        acc[pl.ds(i * 16, 16)] = neg
        return carry

    lax.fori_loop(0, (CPW + 2) * OUT // 16, init_body, 0)

    astart = (start // 8) * 8
    ntiles = (end - astart + TA - 1) // TA

    def row0_of(t):
        return jnp.minimum(astart + t * TA, N - TA)

    def stage(t):
        b = lax.rem(t, 2)
        row0 = row0_of(t)
        pltpu.async_copy(cl_hbm.at[pl.ds(row0, TA)], cl_v.at[b], sem_c.at[b])
        pltpu.async_copy(o_hbm.at[pl.ds(row0, TA), pl.ds(0, OUT)],
                         o_v.at[b], sem_o.at[b])

    def wait_stage(t):
        b = lax.rem(t, 2)
        row0 = row0_of(t)
        pltpu.make_async_copy(cl_hbm.at[pl.ds(row0, TA)], cl_v.at[b],
                              sem_c.at[b]).wait()
        pltpu.make_async_copy(o_hbm.at[pl.ds(row0, TA), pl.ds(0, OUT)],
                              o_v.at[b], sem_o.at[b]).wait()

    def flush(cur_c, ms):
        off = cur_c * OUT
        for j in range(OUT // 16):
            a = acc[pl.ds(off + 16 * j, 16)]
            acc[pl.ds(off + 16 * j, 16)] = jnp.maximum(a, ms[j])

    @pl.when(ntiles > 0)
    def _():
        stage(0)

        def tile_body(t, carry):
            @pl.when(t + 1 < ntiles)
            def _():
                stage(t + 1)

            wait_stage(t)
            b = lax.rem(t, 2)

            def grp_body(g, gc):
                cur_c, ms = gc
                r0 = g * 16
                cv = cl_v[b, pl.ds(r0, 16)]
                clocv = jnp.clip(cv - base, -1, CPW) + 1
                for k in range(16):
                    cloc = clocv[k]
                    vs = [o_v[b, r0 + k, pl.ds(16 * j, 16)]
                          for j in range(OUT // 16)]
                    fl = cloc != cur_c

                    @pl.when(fl)
                    def _(cur_c=cur_c, ms=ms):
                        flush(cur_c, ms)

                    ms = [jnp.where(fl, vs[j], jnp.maximum(ms[j], vs[j]))
                          for j in range(OUT // 16)]
                    cur_c = jnp.where(fl, cloc, cur_c)
                return (cur_c, ms)

            return lax.fori_loop(0, TA // 16, grp_body, carry)

        carry0 = (jnp.int32(0), [neg for _ in range(OUT // 16)])
        cur_c, ms = lax.fori_loop(0, ntiles, tile_body, carry0)
        flush(cur_c, ms)

    pltpu.sync_copy(acc.at[pl.ds(OUT, CPW * OUT)],
                    aggr_hbm.at[pl.ds(base * OUT, CPW * OUT)])


def _sc_segmax(o_flat, cl, bnd):
    mesh = plsc.VectorSubcoreMesh(core_axis_name="c", subcore_axis_name="s",
                                  num_cores=NC, num_subcores=NS)
    f = functools.partial(
        pl.kernel,
        out_type=jax.ShapeDtypeStruct((C * OUT,), jnp.float32),
        mesh=mesh,
        scratch_types=[
            pltpu.VMEM((128,), jnp.int32),
            pltpu.VMEM((2, TA), jnp.int32),
            pltpu.VMEM((2, TA, OUT), jnp.float32),
            pltpu.VMEM(((CPW + 2) * OUT,), jnp.float32),
            pltpu.SemaphoreType.DMA((2,)),
            pltpu.SemaphoreType.DMA((2,)),
        ],
        compiler_params=pltpu.CompilerParams(use_tc_tiling_on_sc=False),
    )(_segmax_body)
    return f(o_flat, cl, bnd)


def _rsqrt16(ss):
    i = lax.bitcast_convert_type(ss, jnp.int32)
    i = jnp.int32(0x5F3759DF) - lax.shift_right_logical(i, 1)
    y = lax.bitcast_convert_type(i, jnp.float32)
    for _ in range(2):
        y = y * (1.5 - 0.5 * ss * y * y)
    return y


def _gather_norm_body(aggr_hbm, o_hbm, cl_hbm, out_hbm, idx_v, g_v, o_v,
                      sem_c, sem_g, sem_o, sem_w):
    cid = lax.axis_index("c")
    sid = lax.axis_index("s")
    w = sid * NC + cid
    nt = N // TB
    nloop = (nt - w + NW - 1) // NW

    def row0_of(t):
        return (w + t * NW) * TB

    def fire_cl(t):
        b = lax.rem(t, 2)
        row0 = row0_of(t)
        for j in range(TB // 128):
            pltpu.async_copy(cl_hbm.at[pl.ds(row0 + 128 * j, 128)],
                             idx_v.at[b, j], sem_c.at[b])

    def wait_cl(t):
        b = lax.rem(t, 2)
        row0 = row0_of(t)
        for j in range(TB // 128):
            pltpu.make_async_copy(cl_hbm.at[pl.ds(row0 + 128 * j, 128)],
                                  idx_v.at[b, j], sem_c.at[b]).wait()

    def fire_go(t):
        b = lax.rem(t, 2)
        row0 = row0_of(t)

        @pl.when(t >= 2)
        def _():
            r2 = row0_of(t - 2)
            pltpu.make_async_copy(
                o_v.at[b], out_hbm.at[pl.ds(r2, TB), pl.ds(0, OUT)],
                sem_w.at[b]).wait()
            pltpu.make_async_copy(
                g_v.at[b], out_hbm.at[pl.ds(r2, TB), pl.ds(OUT, OUT)],
                sem_w.at[b]).wait()

        for j in range(TB // 128):
            pltpu.async_copy(aggr_hbm.at[idx_v.at[b, j]],
                             g_v.at[b, pl.ds(128 * j, 128)], sem_g.at[b])
        pltpu.async_copy(o_hbm.at[pl.ds(row0, TB), pl.ds(0, OUT)],
                         o_v.at[b], sem_o.at[b])

    def wait_go(t):
        b = lax.rem(t, 2)
        row0 = row0_of(t)
        for j in range(TB // 128):
            pltpu.make_async_copy(
                aggr_hbm.at[idx_v.at[b, j]],
                g_v.at[b, pl.ds(128 * j, 128)], sem_g.at[b]).wait()
        pltpu.make_async_copy(o_hbm.at[pl.ds(row0, TB), pl.ds(0, OUT)],
                              o_v.at[b], sem_o.at[b]).wait()

    def compute(t):
        b = lax.rem(t, 2)
        row0 = row0_of(t)

        def grp_body(g, rcarry):
            r0 = g * 4
            for k in range(4):
                r = r0 + k
                ov = [o_v[b, r, pl.ds(16 * j, 16)] for j in range(OUT // 16)]
                gv = [g_v[b, r, pl.ds(16 * j, 16)] for j in range(OUT // 16)]
                p = ov[0] * ov[0]
                for j in range(1, OUT // 16):
                    p = p + ov[j] * ov[j]
                for j in range(OUT // 16):
                    p = p + gv[j] * gv[j]
                lanes = lax.iota(jnp.int32, 16)
                for step in (1, 2, 4, 8):
                    p = p + p.at[lanes ^ step].get(mode="promise_in_bounds")
                ss = jnp.maximum(p, 1e-24)
                y = _rsqrt16(ss)
                for j in range(OUT // 16):
                    o_v[b, r, pl.ds(16 * j, 16)] = ov[j] * y
                    g_v[b, r, pl.ds(16 * j, 16)] = gv[j] * y
            return rcarry

        lax.fori_loop(0, TB // 4, grp_body, 0)
        pltpu.async_copy(o_v.at[b], out_hbm.at[pl.ds(row0, TB), pl.ds(0, OUT)],
                         sem_w.at[b])
        pltpu.async_copy(g_v.at[b],
                         out_hbm.at[pl.ds(row0, TB), pl.ds(OUT, OUT)],
                         sem_w.at[b])

    @pl.when(nloop > 0)
    def _():
        fire_cl(0)

        @pl.when(nloop > 1)
        def _():
            fire_cl(1)

        wait_cl(0)
        fire_go(0)

        def body(t, carry):
            wait_go(t)

            @pl.when(t + 2 < nloop)
            def _():
                fire_cl(t + 2)

            @pl.when(t + 1 < nloop)
            def _():
                wait_cl(t + 1)
                fire_go(t + 1)

            compute(t)
            return carry

        lax.fori_loop(0, nloop, body, 0)

        def drain(t):
            b = lax.rem(t, 2)
            pltpu.make_async_copy(
                o_v.at[b], out_hbm.at[pl.ds(row0_of(t), TB), pl.ds(0, OUT)],
                sem_w.at[b]).wait()
            pltpu.make_async_copy(
                g_v.at[b],
                out_hbm.at[pl.ds(row0_of(t), TB), pl.ds(OUT, OUT)],
                sem_w.at[b]).wait()

        @pl.when(nloop > 1)
        def _():
            drain(nloop - 2)

        drain(nloop - 1)


def _sc_gather_norm(aggr, o2, cl):
    mesh = plsc.VectorSubcoreMesh(core_axis_name="c", subcore_axis_name="s",
                                  num_cores=NC, num_subcores=NS)
    f = functools.partial(
        pl.kernel,
        out_type=jax.ShapeDtypeStruct((N, 2 * OUT), jnp.float32),
        mesh=mesh,
        scratch_types=[
            pltpu.VMEM((2, TB // 128, 128), jnp.int32),
            pltpu.VMEM((2, TB, OUT), jnp.float32),
            pltpu.VMEM((2, TB, OUT), jnp.float32),
            pltpu.SemaphoreType.DMA((2,)),
            pltpu.SemaphoreType.DMA((2,)),
            pltpu.SemaphoreType.DMA((2,)),
            pltpu.SemaphoreType.DMA((2,)),
        ],
        compiler_params=pltpu.CompilerParams(use_tc_tiling_on_sc=False),
    )(_gather_norm_body)
    return f(aggr, o2, cl)


def kernel(x, clusters, batch, W1, b1, W2, b2, gamma, beta):
    del batch
    x = x.astype(jnp.float32)
    cl = clusters.astype(jnp.int32)
    cl3d = cl.reshape(N // RB, 1, RB)
    o2, bnd = _tc_mlp_ln(x, cl3d, W1, b1.reshape(1, H), W2,
                         b2.reshape(1, OUT), gamma.reshape(1, OUT),
                         beta.reshape(1, OUT))
    aggr_flat = _sc_segmax(o2, cl, bnd.reshape(128))
    return _sc_gather_norm(aggr_flat.reshape(C, OUT), o2, cl)


# speedup vs baseline: 1.7861x; 1.0395x over previous
import functools

import jax
import jax.numpy as jnp
from jax import lax
from jax.experimental import pallas as pl
from jax.experimental.pallas import tpu as pltpu
from jax.experimental.pallas import tpu_sc as plsc

N = 320000
D = 128
H = 128
OUT = 64
C = 16000

NC = 2
NS = 16
NW = NC * NS
CPW = C // NW

RB = 4000
TA = 512
TB = 256

_NEG = -3.0e38


def _mlp_ln_body(x_ref, cl_ref, w1_ref, b1_ref, w2_ref, b2_ref, g_ref,
                 be_ref, o_ref, bnd_ref, cnt_ref):
    i = pl.program_id(0)

    @pl.when(i == 0)
    def _():
        cnt_ref[...] = jnp.zeros_like(cnt_ref)

    h = jnp.dot(x_ref[...], w1_ref[...], preferred_element_type=jnp.float32)
    h = h + b1_ref[...]
    h = h * jax.nn.sigmoid(h)
    o = jnp.dot(h, w2_ref[...], preferred_element_type=jnp.float32)
    o = o + b2_ref[...]
    ones_w = jnp.full((OUT, 128), 1.0 / OUT, jnp.float32)
    mu = jnp.dot(o, ones_w, preferred_element_type=jnp.float32)[:, :OUT]
    e2 = jnp.dot(o * o, ones_w, preferred_element_type=jnp.float32)[:, :OUT]
    var = e2 - mu * mu
    o = (o - mu) * jax.lax.rsqrt(var + 1e-5) * g_ref[...] + be_ref[...]
    o_ref[...] = jnp.concatenate([o, jnp.zeros_like(o)], axis=1)

    cl = cl_ref[0, 0, :]
    thr = lax.broadcasted_iota(jnp.int32, (1, 128), 1) * CPW
    cmp = (cl[:, None] < thr).astype(jnp.int32)
    cnt_ref[...] += jnp.sum(cmp, axis=0, keepdims=True)

    @pl.when(i == pl.num_programs(0) - 1)
    def _():
        bnd_ref[...] = cnt_ref[...]


def _tc_mlp_ln(x, cl3d, W1, b1, W2, b2, gamma, beta):
    grid = N // RB
    return pl.pallas_call(
        _mlp_ln_body,
        grid=(grid,),
        in_specs=[
            pl.BlockSpec((RB, D), lambda i: (i, 0)),
            pl.BlockSpec((1, 1, RB), lambda i: (i, 0, 0)),
            pl.BlockSpec((D, H), lambda i: (0, 0)),
            pl.BlockSpec((1, H), lambda i: (0, 0)),
            pl.BlockSpec((H, OUT), lambda i: (0, 0)),
            pl.BlockSpec((1, OUT), lambda i: (0, 0)),
            pl.BlockSpec((1, OUT), lambda i: (0, 0)),
            pl.BlockSpec((1, OUT), lambda i: (0, 0)),
        ],
        out_specs=[
            pl.BlockSpec((RB, 2 * OUT), lambda i: (i, 0)),
            pl.BlockSpec((1, 128), lambda i: (0, 0)),
        ],
        out_shape=[
            jax.ShapeDtypeStruct((N, 2 * OUT), jnp.float32),
            jax.ShapeDtypeStruct((1, 128), jnp.int32),
        ],
        scratch_shapes=[pltpu.VMEM((1, 128), jnp.int32)],
    )(x, cl3d, W1, b1, W2, b2, gamma, beta)


def _segmax_body(o_hbm, cl_hbm, bnd_hbm, aggr_hbm, bnd_v, cl_v, o_v, acc,
                 sem_c, sem_o):
    cid = lax.axis_index("c")
    sid = lax.axis_index("s")
    w = sid * NC + cid
    base = w * CPW

    pltpu.sync_copy(bnd_hbm, bnd_v)
    bv = bnd_v[pl.ds(w, 16)]
    start = bv[0]
    end = bv[1]

    neg = jnp.full((16,), _NEG, jnp.float32)

    def init_body(i, carry):
        acc[pl.ds(i * 16, 16)] = neg
        return carry

    lax.fori_loop(0, (CPW + 2) * OUT // 16, init_body, 0)

    astart = (start // 8) * 8
    ntiles = (end - astart + TA - 1) // TA

    def row0_of(t):
        return jnp.minimum(astart + t * TA, N - TA)

    def stage(t):
        b = lax.rem(t, 2)
        row0 = row0_of(t)
        pltpu.async_copy(cl_hbm.at[pl.ds(row0, TA)], cl_v.at[b], sem_c.at[b])
        pltpu.async_copy(o_hbm.at[pl.ds(row0, TA), pl.ds(0, OUT)],
                         o_v.at[b], sem_o.at[b])

    def wait_stage(t):
        b = lax.rem(t, 2)
        row0 = row0_of(t)
        pltpu.make_async_copy(cl_hbm.at[pl.ds(row0, TA)], cl_v.at[b],
                              sem_c.at[b]).wait()
        pltpu.make_async_copy(o_hbm.at[pl.ds(row0, TA), pl.ds(0, OUT)],
                              o_v.at[b], sem_o.at[b]).wait()

    def flush(cur_c, ms):
        off = cur_c * OUT
        for j in range(OUT // 16):
            a = acc[pl.ds(off + 16 * j, 16)]
            acc[pl.ds(off + 16 * j, 16)] = jnp.maximum(a, ms[j])

    @pl.when(ntiles > 0)
    def _():
        stage(0)

        def tile_body(t, carry):
            @pl.when(t + 1 < ntiles)
            def _():
                stage(t + 1)

            wait_stage(t)
            b = lax.rem(t, 2)

            def grp_body(g, gc):
                cur_c, ms = gc
                r0 = g * 16
                cv = cl_v[b, pl.ds(r0, 16)]
                clocv = jnp.clip(cv - base, -1, CPW) + 1
                for k in range(16):
                    cloc = clocv[k]
                    vs = [o_v[b, r0 + k, pl.ds(16 * j, 16)]
                          for j in range(OUT // 16)]
                    fl = cloc != cur_c

                    @pl.when(fl)
                    def _(cur_c=cur_c, ms=ms):
                        flush(cur_c, ms)

                    ms = [jnp.where(fl, vs[j], jnp.maximum(ms[j], vs[j]))
                          for j in range(OUT // 16)]
                    cur_c = jnp.where(fl, cloc, cur_c)
                return (cur_c, ms)

            return lax.fori_loop(0, TA // 16, grp_body, carry)

        carry0 = (jnp.int32(0), [neg for _ in range(OUT // 16)])
        cur_c, ms = lax.fori_loop(0, ntiles, tile_body, carry0)
        flush(cur_c, ms)

    pltpu.sync_copy(acc.at[pl.ds(OUT, CPW * OUT)],
                    aggr_hbm.at[pl.ds(base * OUT, CPW * OUT)])


def _sc_segmax(o_flat, cl, bnd):
    mesh = plsc.VectorSubcoreMesh(core_axis_name="c", subcore_axis_name="s",
                                  num_cores=NC, num_subcores=NS)
    f = functools.partial(
        pl.kernel,
        out_type=jax.ShapeDtypeStruct((C * OUT,), jnp.float32),
        mesh=mesh,
        scratch_types=[
            pltpu.VMEM((128,), jnp.int32),
            pltpu.VMEM((2, TA), jnp.int32),
            pltpu.VMEM((2, TA, OUT), jnp.float32),
            pltpu.VMEM(((CPW + 2) * OUT,), jnp.float32),
            pltpu.SemaphoreType.DMA((2,)),
            pltpu.SemaphoreType.DMA((2,)),
        ],
        compiler_params=pltpu.CompilerParams(use_tc_tiling_on_sc=False),
    )(_segmax_body)
    return f(o_flat, cl, bnd)


def _rsqrt16(ss):
    i = lax.bitcast_convert_type(ss, jnp.int32)
    i = jnp.int32(0x5F3759DF) - lax.shift_right_logical(i, 1)
    y = lax.bitcast_convert_type(i, jnp.float32)
    for _ in range(2):
        y = y * (1.5 - 0.5 * ss * y * y)
    return y


def _gather_norm_body(aggr_hbm, o_hbm, cl_hbm, out_hbm, idx_v, g_v, o_v,
                      sem_c, sem_g, sem_o, sem_w):
    cid = lax.axis_index("c")
    sid = lax.axis_index("s")
    w = sid * NC + cid
    nt = N // TB
    nloop = (nt - w + NW - 1) // NW

    def row0_of(t):
        return (w + t * NW) * TB

    def fire_cl(t):
        b = lax.rem(t, 2)
        row0 = row0_of(t)
        for j in range(TB // 128):
            pltpu.async_copy(cl_hbm.at[pl.ds(row0 + 128 * j, 128)],
                             idx_v.at[b, j], sem_c.at[b])

    def wait_cl(t):
        b = lax.rem(t, 2)
        row0 = row0_of(t)
        for j in range(TB // 128):
            pltpu.make_async_copy(cl_hbm.at[pl.ds(row0 + 128 * j, 128)],
                                  idx_v.at[b, j], sem_c.at[b]).wait()

    def fire_go(t):
        b = lax.rem(t, 2)
        row0 = row0_of(t)

        @pl.when(t >= 2)
        def _():
            r2 = row0_of(t - 2)
            pltpu.make_async_copy(
                o_v.at[b], out_hbm.at[pl.ds(r2, TB), pl.ds(0, OUT)],
                sem_w.at[b]).wait()
            pltpu.make_async_copy(
                g_v.at[b], out_hbm.at[pl.ds(r2, TB), pl.ds(OUT, OUT)],
                sem_w.at[b]).wait()

        for j in range(TB // 128):
            pltpu.async_copy(aggr_hbm.at[idx_v.at[b, j]],
                             g_v.at[b, pl.ds(128 * j, 128)], sem_g.at[b])
        pltpu.async_copy(o_hbm.at[pl.ds(row0, TB), pl.ds(0, OUT)],
                         o_v.at[b], sem_o.at[b])

    def wait_go(t):
        b = lax.rem(t, 2)
        row0 = row0_of(t)
        for j in range(TB // 128):
            pltpu.make_async_copy(
                aggr_hbm.at[idx_v.at[b, j]],
                g_v.at[b, pl.ds(128 * j, 128)], sem_g.at[b]).wait()
        pltpu.make_async_copy(o_hbm.at[pl.ds(row0, TB), pl.ds(0, OUT)],
                              o_v.at[b], sem_o.at[b]).wait()

    def compute(t):
        b = lax.rem(t, 2)
        row0 = row0_of(t)

        def grp_body(g, rcarry):
            r0 = g * 4
            for k in range(4):
                r = r0 + k
                ov = [o_v[b, r, pl.ds(16 * j, 16)] for j in range(OUT // 16)]
                gv = [g_v[b, r, pl.ds(16 * j, 16)] for j in range(OUT // 16)]
                p = ov[0] * ov[0]
                for j in range(1, OUT // 16):
                    p = p + ov[j] * ov[j]
                for j in range(OUT // 16):
                    p = p + gv[j] * gv[j]
                lanes = lax.iota(jnp.int32, 16)
                for step in (1, 2, 4, 8):
                    p = p + p.at[lanes ^ step].get(mode="promise_in_bounds")
                ss = jnp.maximum(p, 1e-24)
                y = _rsqrt16(ss)
                for j in range(OUT // 16):
                    o_v[b, r, pl.ds(16 * j, 16)] = ov[j] * y
                    g_v[b, r, pl.ds(16 * j, 16)] = gv[j] * y
            return rcarry

        lax.fori_loop(0, TB // 4, grp_body, 0)
        pltpu.async_copy(o_v.at[b], out_hbm.at[pl.ds(row0, TB), pl.ds(0, OUT)],
                         sem_w.at[b])
        pltpu.async_copy(g_v.at[b],
                         out_hbm.at[pl.ds(row0, TB), pl.ds(OUT, OUT)],
                         sem_w.at[b])

    @pl.when(nloop > 0)
    def _():
        fire_cl(0)

        @pl.when(nloop > 1)
        def _():
            fire_cl(1)

        wait_cl(0)
        fire_go(0)

        def body(t, carry):
            wait_go(t)

            @pl.when(t + 2 < nloop)
            def _():
                fire_cl(t + 2)

            @pl.when(t + 1 < nloop)
            def _():
                wait_cl(t + 1)
                fire_go(t + 1)

            compute(t)
            return carry

        lax.fori_loop(0, nloop, body, 0)

        def drain(t):
            b = lax.rem(t, 2)
            pltpu.make_async_copy(
                o_v.at[b], out_hbm.at[pl.ds(row0_of(t), TB), pl.ds(0, OUT)],
                sem_w.at[b]).wait()
            pltpu.make_async_copy(
                g_v.at[b],
                out_hbm.at[pl.ds(row0_of(t), TB), pl.ds(OUT, OUT)],
                sem_w.at[b]).wait()

        @pl.when(nloop > 1)
        def _():
            drain(nloop - 2)

        drain(nloop - 1)


def _sc_gather_norm(aggr, o2, cl):
    mesh = plsc.VectorSubcoreMesh(core_axis_name="c", subcore_axis_name="s",
                                  num_cores=NC, num_subcores=NS)
    f = functools.partial(
        pl.kernel,
        out_type=jax.ShapeDtypeStruct((N, 2 * OUT), jnp.float32),
        mesh=mesh,
        scratch_types=[
            pltpu.VMEM((2, TB // 128, 128), jnp.int32),
            pltpu.VMEM((2, TB, OUT), jnp.float32),
            pltpu.VMEM((2, TB, OUT), jnp.float32),
            pltpu.SemaphoreType.DMA((2,)),
            pltpu.SemaphoreType.DMA((2,)),
            pltpu.SemaphoreType.DMA((2,)),
            pltpu.SemaphoreType.DMA((2,)),
        ],
        compiler_params=pltpu.CompilerParams(use_tc_tiling_on_sc=False),
    )(_gather_norm_body)
    return f(aggr, o2, cl)


def kernel(x, clusters, batch, W1, b1, W2, b2, gamma, beta):
    del batch
    x = x.astype(jnp.float32)
    cl = clusters.astype(jnp.int32)
    cl3d = cl.reshape(N // RB, 1, RB)
    o2, bnd = _tc_mlp_ln(x, cl3d, W1, b1.reshape(1, H), W2,
                         b2.reshape(1, OUT), gamma.reshape(1, OUT),
                         beta.reshape(1, OUT))
    aggr_flat = _sc_segmax(o2, cl, bnd.reshape(128))
    return _sc_gather_norm(aggr_flat.reshape(C, OUT), o2, cl)


# speedup vs baseline: 1.8517x; 1.0367x over previous
import functools

import jax
import jax.numpy as jnp
from jax import lax
from jax.experimental import pallas as pl
from jax.experimental.pallas import tpu as pltpu
from jax.experimental.pallas import tpu_sc as plsc

N = 320000
D = 128
H = 128
OUT = 64
C = 16000

NC = 2
NS = 16
NW = NC * NS
CPW = C // NW

RB = 8000
TA = 512
TB = 256

_NEG = -3.0e38


def _mlp_ln_body(x_ref, cl_ref, w1_ref, b1_ref, w2_ref, b2_ref, g_ref,
                 be_ref, o_ref, bnd_ref, cnt_ref):
    i = pl.program_id(0)

    @pl.when(i == 0)
    def _():
        cnt_ref[...] = jnp.zeros_like(cnt_ref)

    h = jnp.dot(x_ref[...], w1_ref[...], preferred_element_type=jnp.float32)
    h = h + b1_ref[...]
    h = h * jax.nn.sigmoid(h)
    o = jnp.dot(h, w2_ref[...], preferred_element_type=jnp.float32)
    o = o + b2_ref[...]
    ones_w = jnp.full((OUT, 128), 1.0 / OUT, jnp.float32)
    mu = jnp.dot(o, ones_w, preferred_element_type=jnp.float32)[:, :OUT]
    e2 = jnp.dot(o * o, ones_w, preferred_element_type=jnp.float32)[:, :OUT]
    var = e2 - mu * mu
    o = (o - mu) * jax.lax.rsqrt(var + 1e-5) * g_ref[...] + be_ref[...]
    o_ref[...] = jnp.concatenate([o, jnp.zeros_like(o)], axis=1)

    cl = cl_ref[0, 0, :]
    thr = lax.broadcasted_iota(jnp.int32, (1, 128), 1) * CPW
    cmp = (cl[:, None] < thr).astype(jnp.int32)
    cnt_ref[...] += jnp.sum(cmp, axis=0, keepdims=True)

    @pl.when(i == pl.num_programs(0) - 1)
    def _():
        bnd_ref[...] = cnt_ref[...]


def _tc_mlp_ln(x, cl3d, W1, b1, W2, b2, gamma, beta):
    grid = N // RB
    return pl.pallas_call(
        _mlp_ln_body,
        grid=(grid,),
        in_specs=[
            pl.BlockSpec((RB, D), lambda i: (i, 0)),
            pl.BlockSpec((1, 1, RB), lambda i: (i, 0, 0)),
            pl.BlockSpec((D, H), lambda i: (0, 0)),
            pl.BlockSpec((1, H), lambda i: (0, 0)),
            pl.BlockSpec((H, OUT), lambda i: (0, 0)),
            pl.BlockSpec((1, OUT), lambda i: (0, 0)),
            pl.BlockSpec((1, OUT), lambda i: (0, 0)),
            pl.BlockSpec((1, OUT), lambda i: (0, 0)),
        ],
        out_specs=[
            pl.BlockSpec((RB, 2 * OUT), lambda i: (i, 0)),
            pl.BlockSpec((1, 128), lambda i: (0, 0)),
        ],
        out_shape=[
            jax.ShapeDtypeStruct((N, 2 * OUT), jnp.float32),
            jax.ShapeDtypeStruct((1, 128), jnp.int32),
        ],
        scratch_shapes=[pltpu.VMEM((1, 128), jnp.int32)],
    )(x, cl3d, W1, b1, W2, b2, gamma, beta)


def _segmax_body(o_hbm, cl_hbm, bnd_hbm, aggr_hbm, bnd_v, cl_v, o_v, acc,
                 sem_c, sem_o):
    cid = lax.axis_index("c")
    sid = lax.axis_index("s")
    w = sid * NC + cid
    base = w * CPW

    pltpu.sync_copy(bnd_hbm, bnd_v)
    bv = bnd_v[pl.ds(w, 16)]
    start = bv[0]
    end = bv[1]

    neg = jnp.full((16,), _NEG, jnp.float32)

    def init_body(i, carry):
        acc[pl.ds(i * 16, 16)] = neg
        return carry

    lax.fori_loop(0, (CPW + 2) * OUT // 16, init_body, 0)

    astart = (start // 8) * 8
    ntiles = (end - astart + TA - 1) // TA

    def row0_of(t):
        return jnp.minimum(astart + t * TA, N - TA)

    def stage(t):
        b = lax.rem(t, 2)
        row0 = row0_of(t)
        pltpu.async_copy(cl_hbm.at[pl.ds(row0, TA)], cl_v.at[b], sem_c.at[b])
        pltpu.async_copy(o_hbm.at[pl.ds(row0, TA), pl.ds(0, OUT)],
                         o_v.at[b], sem_o.at[b])

    def wait_stage(t):
        b = lax.rem(t, 2)
        row0 = row0_of(t)
        pltpu.make_async_copy(cl_hbm.at[pl.ds(row0, TA)], cl_v.at[b],
                              sem_c.at[b]).wait()
        pltpu.make_async_copy(o_hbm.at[pl.ds(row0, TA), pl.ds(0, OUT)],
                              o_v.at[b], sem_o.at[b]).wait()

    def flush(cur_c, ms):
        off = cur_c * OUT
        for j in range(OUT // 16):
            a = acc[pl.ds(off + 16 * j, 16)]
            acc[pl.ds(off + 16 * j, 16)] = jnp.maximum(a, ms[j])

    @pl.when(ntiles > 0)
    def _():
        stage(0)

        def tile_body(t, carry):
            @pl.when(t + 1 < ntiles)
            def _():
                stage(t + 1)

            wait_stage(t)
            b = lax.rem(t, 2)

            def grp_body(g, gc):
                cur_c, ms = gc
                r0 = g * 16
                cv = cl_v[b, pl.ds(r0, 16)]
                clocv = jnp.clip(cv - base, -1, CPW) + 1
                for k in range(16):
                    cloc = clocv[k]
                    vs = [o_v[b, r0 + k, pl.ds(16 * j, 16)]
                          for j in range(OUT // 16)]
                    fl = cloc != cur_c

                    @pl.when(fl)
                    def _(cur_c=cur_c, ms=ms):
                        flush(cur_c, ms)

                    ms = [jnp.where(fl, vs[j], jnp.maximum(ms[j], vs[j]))
                          for j in range(OUT // 16)]
                    cur_c = jnp.where(fl, cloc, cur_c)
                return (cur_c, ms)

            return lax.fori_loop(0, TA // 16, grp_body, carry)

        carry0 = (jnp.int32(0), [neg for _ in range(OUT // 16)])
        cur_c, ms = lax.fori_loop(0, ntiles, tile_body, carry0)
        flush(cur_c, ms)

    pltpu.sync_copy(acc.at[pl.ds(OUT, CPW * OUT)],
                    aggr_hbm.at[pl.ds(base * OUT, CPW * OUT)])


def _sc_segmax(o_flat, cl, bnd):
    mesh = plsc.VectorSubcoreMesh(core_axis_name="c", subcore_axis_name="s",
                                  num_cores=NC, num_subcores=NS)
    f = functools.partial(
        pl.kernel,
        out_type=jax.ShapeDtypeStruct((C * OUT,), jnp.float32),
        mesh=mesh,
        scratch_types=[
            pltpu.VMEM((128,), jnp.int32),
            pltpu.VMEM((2, TA), jnp.int32),
            pltpu.VMEM((2, TA, OUT), jnp.float32),
            pltpu.VMEM(((CPW + 2) * OUT,), jnp.float32),
            pltpu.SemaphoreType.DMA((2,)),
            pltpu.SemaphoreType.DMA((2,)),
        ],
        compiler_params=pltpu.CompilerParams(use_tc_tiling_on_sc=False),
    )(_segmax_body)
    return f(o_flat, cl, bnd)


def _rsqrt16(ss):
    i = lax.bitcast_convert_type(ss, jnp.int32)
    i = jnp.int32(0x5F3759DF) - lax.shift_right_logical(i, 1)
    y = lax.bitcast_convert_type(i, jnp.float32)
    for _ in range(2):
        y = y * (1.5 - 0.5 * ss * y * y)
    return y


def _gather_norm_body(aggr_hbm, o_hbm, cl_hbm, out_hbm, idx_v, g_v, o_v,
                      sem_c, sem_g, sem_o, sem_w):
    cid = lax.axis_index("c")
    sid = lax.axis_index("s")
    w = sid * NC + cid
    nt = N // TB
    nloop = (nt - w + NW - 1) // NW

    def row0_of(t):
        return (w + t * NW) * TB

    def fire_cl(t):
        b = lax.rem(t, 2)
        row0 = row0_of(t)
        for j in range(TB // 128):
            pltpu.async_copy(cl_hbm.at[pl.ds(row0 + 128 * j, 128)],
                             idx_v.at[b, j], sem_c.at[b])

    def wait_cl(t):
        b = lax.rem(t, 2)
        row0 = row0_of(t)
        for j in range(TB // 128):
            pltpu.make_async_copy(cl_hbm.at[pl.ds(row0 + 128 * j, 128)],
                                  idx_v.at[b, j], sem_c.at[b]).wait()

    def fire_go(t):
        b = lax.rem(t, 2)
        row0 = row0_of(t)

        @pl.when(t >= 2)
        def _():
            r2 = row0_of(t - 2)
            pltpu.make_async_copy(
                o_v.at[b], out_hbm.at[pl.ds(r2, TB), pl.ds(0, OUT)],
                sem_w.at[b]).wait()
            pltpu.make_async_copy(
                g_v.at[b], out_hbm.at[pl.ds(r2, TB), pl.ds(OUT, OUT)],
                sem_w.at[b]).wait()

        for j in range(TB // 128):
            pltpu.async_copy(aggr_hbm.at[idx_v.at[b, j]],
                             g_v.at[b, pl.ds(128 * j, 128)], sem_g.at[b])
        pltpu.async_copy(o_hbm.at[pl.ds(row0, TB), pl.ds(0, OUT)],
                         o_v.at[b], sem_o.at[b])

    def wait_go(t):
        b = lax.rem(t, 2)
        row0 = row0_of(t)
        for j in range(TB // 128):
            pltpu.make_async_copy(
                aggr_hbm.at[idx_v.at[b, j]],
                g_v.at[b, pl.ds(128 * j, 128)], sem_g.at[b]).wait()
        pltpu.make_async_copy(o_hbm.at[pl.ds(row0, TB), pl.ds(0, OUT)],
                              o_v.at[b], sem_o.at[b]).wait()

    def compute(t):
        b = lax.rem(t, 2)
        row0 = row0_of(t)

        def grp_body(g, rcarry):
            r0 = g * 4
            for k in range(4):
                r = r0 + k
                ov = [o_v[b, r, pl.ds(16 * j, 16)] for j in range(OUT // 16)]
                gv = [g_v[b, r, pl.ds(16 * j, 16)] for j in range(OUT // 16)]
                p = ov[0] * ov[0]
                for j in range(1, OUT // 16):
                    p = p + ov[j] * ov[j]
                for j in range(OUT // 16):
                    p = p + gv[j] * gv[j]
                lanes = lax.iota(jnp.int32, 16)
                for step in (1, 2, 4, 8):
                    p = p + p.at[lanes ^ step].get(mode="promise_in_bounds")
                ss = jnp.maximum(p, 1e-24)
                y = _rsqrt16(ss)
                for j in range(OUT // 16):
                    o_v[b, r, pl.ds(16 * j, 16)] = ov[j] * y
                    g_v[b, r, pl.ds(16 * j, 16)] = gv[j] * y
            return rcarry

        lax.fori_loop(0, TB // 4, grp_body, 0)
        pltpu.async_copy(o_v.at[b], out_hbm.at[pl.ds(row0, TB), pl.ds(0, OUT)],
                         sem_w.at[b])
        pltpu.async_copy(g_v.at[b],
                         out_hbm.at[pl.ds(row0, TB), pl.ds(OUT, OUT)],
                         sem_w.at[b])

    @pl.when(nloop > 0)
    def _():
        fire_cl(0)

        @pl.when(nloop > 1)
        def _():
            fire_cl(1)

        wait_cl(0)
        fire_go(0)

        def body(t, carry):
            wait_go(t)

            @pl.when(t + 2 < nloop)
            def _():
                fire_cl(t + 2)

            @pl.when(t + 1 < nloop)
            def _():
                wait_cl(t + 1)
                fire_go(t + 1)

            compute(t)
            return carry

        lax.fori_loop(0, nloop, body, 0)

        def drain(t):
            b = lax.rem(t, 2)
            pltpu.make_async_copy(
                o_v.at[b], out_hbm.at[pl.ds(row0_of(t), TB), pl.ds(0, OUT)],
                sem_w.at[b]).wait()
            pltpu.make_async_copy(
                g_v.at[b],
                out_hbm.at[pl.ds(row0_of(t), TB), pl.ds(OUT, OUT)],
                sem_w.at[b]).wait()

        @pl.when(nloop > 1)
        def _():
            drain(nloop - 2)

        drain(nloop - 1)


def _sc_gather_norm(aggr, o2, cl):
    mesh = plsc.VectorSubcoreMesh(core_axis_name="c", subcore_axis_name="s",
                                  num_cores=NC, num_subcores=NS)
    f = functools.partial(
        pl.kernel,
        out_type=jax.ShapeDtypeStruct((N, 2 * OUT), jnp.float32),
        mesh=mesh,
        scratch_types=[
            pltpu.VMEM((2, TB // 128, 128), jnp.int32),
            pltpu.VMEM((2, TB, OUT), jnp.float32),
            pltpu.VMEM((2, TB, OUT), jnp.float32),
            pltpu.SemaphoreType.DMA((2,)),
            pltpu.SemaphoreType.DMA((2,)),
            pltpu.SemaphoreType.DMA((2,)),
            pltpu.SemaphoreType.DMA((2,)),
        ],
        compiler_params=pltpu.CompilerParams(use_tc_tiling_on_sc=False),
    )(_gather_norm_body)
    return f(aggr, o2, cl)


def kernel(x, clusters, batch, W1, b1, W2, b2, gamma, beta):
    del batch
    x = x.astype(jnp.float32)
    cl = clusters.astype(jnp.int32)
    cl3d = cl.reshape(N // RB, 1, RB)
    o2, bnd = _tc_mlp_ln(x, cl3d, W1, b1.reshape(1, H), W2,
                         b2.reshape(1, OUT), gamma.reshape(1, OUT),
                         beta.reshape(1, OUT))
    aggr_flat = _sc_segmax(o2, cl, bnd.reshape(128))
    return _sc_gather_norm(aggr_flat.reshape(C, OUT), o2, cl)
